# Initial kernel scaffold; baseline (speedup 1.0000x reference)
import jax
import jax.numpy as jnp
from jax.experimental import pallas as pl


def kernel(x):
    raise NotImplementedError("write your pallas kernel here")


# speedup vs baseline: 1.0116x; 1.0116x over previous
import functools

import jax
import jax.numpy as jnp
from jax import lax
from jax.experimental import pallas as pl
from jax.experimental.pallas import tpu as pltpu
from jax.experimental.pallas import tpu_sc as plsc

_CORRUPTION_RATE = 0.6

_NUM_CORES = 2
_NUM_SUBCORES = 16
_NUM_WORKERS = _NUM_CORES * _NUM_SUBCORES


@functools.lru_cache(maxsize=None)
def _gather_indices(B: int, S: int, F: int):
    N = B * S
    key = jax.random.key(42)
    kmask, kperm = jax.random.split(key)
    mask = jax.random.uniform(kmask, (F,)) < _CORRUPTION_RATE
    u = jax.random.uniform(kperm, (F, N))
    perm = jnp.argsort(u, axis=1)
    src = jnp.where(mask[:, None], perm, jnp.arange(N, dtype=perm.dtype)[None, :])
    gidx = (src.T * F + jnp.arange(F, dtype=perm.dtype)[None, :]).reshape(-1)
    return gidx.astype(jnp.int32)


@functools.lru_cache(maxsize=None)
def _build_sc_gather(total: int, chunk: int):
    per_w = total // _NUM_WORKERS
    n_chunks = per_w // chunk
    mesh = plsc.VectorSubcoreMesh(core_axis_name="c", subcore_axis_name="s")

    @functools.partial(
        pl.kernel,
        mesh=mesh,
        out_type=jax.ShapeDtypeStruct((total,), jnp.float32),
        scratch_types=[
            pltpu.VMEM((chunk,), jnp.int32),
            pltpu.VMEM((chunk,), jnp.float32),
            pltpu.SemaphoreType.DMA,
        ],
    )
    def sc_gather(x_hbm, gidx_hbm, out_hbm, idx_v, val_v, sem):
        wid = lax.axis_index("s") * _NUM_CORES + lax.axis_index("c")
        base = wid * per_w

        def body(c, carry):
            off = base + c * chunk
            pltpu.sync_copy(gidx_hbm.at[pl.ds(off, chunk)], idx_v)
            pltpu.async_copy(x_hbm.at[idx_v], val_v, sem).wait()
            pltpu.sync_copy(val_v, out_hbm.at[pl.ds(off, chunk)])
            return carry

        lax.fori_loop(0, n_chunks, body, 0)

    return sc_gather


def kernel(x):
    B, S, F = x.shape
    total = B * S * F
    gidx = _gather_indices(B, S, F)
    per_w = total // _NUM_WORKERS
    chunk = 25600
    while per_w % chunk:
        chunk //= 2
    out = _build_sc_gather(total, chunk)(x.reshape(-1), gidx)
    return out.reshape(B, S, F)


# speedup vs baseline: 14.4930x; 14.3273x over previous
import functools

import jax
import jax.numpy as jnp
import numpy as np
from jax import lax
from jax.experimental import pallas as pl
from jax.experimental.pallas import tpu as pltpu
from jax.experimental.pallas import tpu_sc as plsc

_CORRUPTION_RATE = 0.6

_NUM_CORES = 2
_NUM_SUBCORES = 16
_NW = _NUM_CORES * _NUM_SUBCORES

_A = 40960
_BK = 512
_SEG = 128
_SEG_BITS = 7
_SUB = 8192
_SUB2 = 10240


def _threefry_pair(k0, k1, x0, x1):
    old = np.seterr(over="ignore")
    try:
        x0 = x0.astype(np.uint32).copy()
        x1 = x1.astype(np.uint32).copy()
        ks = [
            np.uint32(k0),
            np.uint32(k1),
            np.uint32(np.uint32(k0) ^ np.uint32(k1) ^ np.uint32(0x1BD11BDA)),
        ]
        rot0 = (13, 15, 26, 6)
        rot1 = (17, 29, 16, 24)

        def rotl(x, r):
            return (x << np.uint32(r)) | (x >> np.uint32(32 - r))

        x0 += ks[0]
        x1 += ks[1]
        for i in range(5):
            for r in rot0 if i % 2 == 0 else rot1:
                x0 += x1
                x1 = rotl(x1, r)
                x1 ^= x0
            x0 += ks[(i + 1) % 3]
            x1 += ks[(i + 2) % 3] + np.uint32(i + 1)
        return x0, x1
    finally:
        np.seterr(**old)


def _np_uniform(k0, k1, n):
    hi = np.zeros(n, np.uint32)
    lo = np.arange(n, dtype=np.uint32)
    b1, b2 = _threefry_pair(k0, k1, hi, lo)
    bits = b1 ^ b2
    f = ((bits >> np.uint32(9)) | np.uint32(0x3F800000)).view(np.float32)
    return np.maximum(np.float32(0.0), f - np.float32(1.0))


@functools.lru_cache(maxsize=None)
def _flat_perm(B: int, S: int, F: int):
    N = B * S
    (m0, m1), (p0, p1) = zip(*_threefry_pair(
        np.uint32(0), np.uint32(42),
        np.zeros(2, np.uint32), np.arange(2, dtype=np.uint32)))
    mask = _np_uniform(m0, m1, F) < _CORRUPTION_RATE
    u = _np_uniform(p0, p1, F * N).reshape(F, N)
    perm = np.argsort(u, axis=1, kind="stable").astype(np.int64)
    src = np.where(mask[:, None], perm, np.arange(N, dtype=np.int64)[None, :])
    g = (src.T * F + np.arange(F, dtype=np.int64)[None, :]).reshape(-1)
    return g.astype(np.int32)


@functools.lru_cache(maxsize=None)
def _twopass_indices(B: int, S: int, F: int):
    g = _flat_perm(B, S, F).astype(np.int64)
    M = g.size
    bucket_size = M // _BK
    nc1 = M // _A
    h = np.empty(M, dtype=np.int64)
    h[g] = np.arange(M, dtype=np.int64)
    b_of_p = h // bucket_size
    s_of_p = np.arange(M, dtype=np.int64) // _A
    cell = s_of_p * _BK + b_of_p
    order = np.argsort(cell, kind="stable")
    counts = np.bincount(cell, minlength=nc1 * _BK)
    starts = np.zeros(nc1 * _BK, dtype=np.int64)
    starts[1:] = np.cumsum(counts)[:-1]
    rank = np.empty(M, dtype=np.int64)
    rank[order] = np.arange(M, dtype=np.int64) - starts[cell[order]]
    s1 = (b_of_p * _SEG + rank).astype(np.int32)
    i2 = (s_of_p[g] * _SEG + rank[g]).astype(np.int32)
    return s1, i2, int(counts.max())


@functools.lru_cache(maxsize=None)
def _build_pass1(M: int):
    nc1 = M // _A
    cpt = nc1 // _NW
    mesh = plsc.VectorSubcoreMesh(core_axis_name="c", subcore_axis_name="s")

    @functools.partial(
        pl.kernel,
        mesh=mesh,
        out_type=jax.ShapeDtypeStruct((nc1, _BK, _SEG), jnp.float32),
        scratch_types=[
            pltpu.VMEM((_A,), jnp.float32),
            pltpu.VMEM((_SUB,), jnp.int32),
            pltpu.VMEM((_BK, _SEG), jnp.float32),
        ],
    )
    def pass1(x_hbm, s1_hbm, interm_hbm, src_v, sidx_v, stg_v):
        wid = lax.axis_index("s") * _NUM_CORES + lax.axis_index("c")

        def chunk_body(ci, carry):
            s = wid * cpt + ci
            base = s * _A
            pltpu.sync_copy(x_hbm.at[pl.ds(base, _A)], src_v)

            def sub_body(k, carry):
                pltpu.sync_copy(s1_hbm.at[pl.ds(base + k * _SUB, _SUB)], sidx_v)

                def vec_body(t, carry):
                    idx = sidx_v[pl.ds(t * 16, 16)]
                    val = src_v[pl.ds(k * _SUB + t * 16, 16)]
                    plsc.store_scatter(
                        stg_v, [idx >> _SEG_BITS, idx & (_SEG - 1)], val
                    )
                    return carry

                return lax.fori_loop(0, _SUB // 16, vec_body, carry)

            lax.fori_loop(0, _A // _SUB, sub_body, carry)
            pltpu.sync_copy(stg_v, interm_hbm.at[s])
            return carry

        lax.fori_loop(0, cpt, chunk_body, 0)

    return pass1


@functools.lru_cache(maxsize=None)
def _build_pass2(M: int):
    nc1 = M // _A
    bucket_size = M // _BK
    bpt = _BK // _NW
    mesh = plsc.VectorSubcoreMesh(core_axis_name="c", subcore_axis_name="s")

    @functools.partial(
        pl.kernel,
        mesh=mesh,
        out_type=jax.ShapeDtypeStruct((M,), jnp.float32),
        scratch_types=[
            pltpu.VMEM((nc1, _SEG), jnp.float32),
            pltpu.VMEM((_SUB2,), jnp.int32),
            pltpu.VMEM((_SUB2,), jnp.float32),
        ],
    )
    def pass2(interm_hbm, i2_hbm, out_hbm, bucket_v, oidx_v, out_v):
        wid = lax.axis_index("s") * _NUM_CORES + lax.axis_index("c")

        def bucket_body(bi, carry):
            b = wid * bpt + bi
            pltpu.sync_copy(interm_hbm.at[:, b, :], bucket_v)
            obase = b * bucket_size

            def sub_body(k, carry):
                off = obase + k * _SUB2
                pltpu.sync_copy(i2_hbm.at[pl.ds(off, _SUB2)], oidx_v)

                def vec_body(t, carry):
                    idx = oidx_v[pl.ds(t * 16, 16)]
                    out_v[pl.ds(t * 16, 16)] = plsc.load_gather(
                        bucket_v, [idx >> _SEG_BITS, idx & (_SEG - 1)]
                    )
                    return carry

                lax.fori_loop(0, _SUB2 // 16, vec_body, carry)
                pltpu.sync_copy(out_v, out_hbm.at[pl.ds(off, _SUB2)])
                return carry

            return lax.fori_loop(0, bucket_size // _SUB2, sub_body, carry)

        lax.fori_loop(0, bpt, bucket_body, 0)

    return pass2


@functools.lru_cache(maxsize=None)
def _build_sc_gather(total: int, chunk: int):
    per_w = total // _NW
    n_chunks = per_w // chunk
    mesh = plsc.VectorSubcoreMesh(core_axis_name="c", subcore_axis_name="s")

    @functools.partial(
        pl.kernel,
        mesh=mesh,
        out_type=jax.ShapeDtypeStruct((total,), jnp.float32),
        scratch_types=[
            pltpu.VMEM((chunk,), jnp.int32),
            pltpu.VMEM((chunk,), jnp.float32),
            pltpu.SemaphoreType.DMA,
        ],
    )
    def sc_gather(x_hbm, gidx_hbm, out_hbm, idx_v, val_v, sem):
        wid = lax.axis_index("s") * _NUM_CORES + lax.axis_index("c")
        base = wid * per_w

        def body(c, carry):
            off = base + c * chunk
            pltpu.sync_copy(gidx_hbm.at[pl.ds(off, chunk)], idx_v)
            pltpu.async_copy(x_hbm.at[idx_v], val_v, sem).wait()
            pltpu.sync_copy(val_v, out_hbm.at[pl.ds(off, chunk)])
            return carry

        lax.fori_loop(0, n_chunks, body, 0)

    return sc_gather


def _twopass_ok(M: int) -> bool:
    if M % _A or (M // _A) % _NW or M % _BK:
        return False
    bucket_size = M // _BK
    return _A % _SUB == 0 and bucket_size % _SUB2 == 0 and _BK % _NW == 0


def kernel(x):
    B, S, F = x.shape
    M = B * S * F
    x1d = x.reshape(-1)
    if _twopass_ok(M):
        s1, i2, max_count = _twopass_indices(B, S, F)
        if max_count <= _SEG:
            interm = _build_pass1(M)(x1d, s1)
            out = _build_pass2(M)(interm, i2)
            return out.reshape(B, S, F)
    gidx = _flat_perm(B, S, F)
    per_w = M // _NW
    chunk = 25600
    while per_w % chunk:
        chunk //= 2
    out = _build_sc_gather(M, chunk)(x1d, gidx)
    return out.reshape(B, S, F)
